# baseline (speedup 1.0000x reference)
import functools

import jax
import jax.numpy as jnp
from jax import lax
from jax.experimental import pallas as pl
from jax.experimental.pallas import tpu as pltpu
from jax.experimental.pallas import tpu_sc as plsc

N_ANG = 360
LANES = 16
NCHUNK = 23
NPAD = NCHUNK * LANES
TAU_INV = 0.2
LN2 = 0.6931471805599453
SQRT2 = 1.4142135


def _log16(x):
    bits = plsc.bitcast(x, jnp.int32)
    e = ((bits >> 23) & 0xFF) - 127
    m = plsc.bitcast((bits & 0x007FFFFF) | (127 << 23), jnp.float32)
    adj = m > SQRT2
    m = jnp.where(adj, m * 0.5, m)
    e = jnp.where(adj, e + 1, e)
    s = (m - 1.0) / (m + 1.0)
    s2 = s * s
    p = (1.0 / 3.0) + s2 * ((1.0 / 5.0) + s2 * ((1.0 / 7.0) + s2 * (1.0 / 9.0)))
    atanh = s + s * s2 * p
    return e.astype(jnp.float32) * LN2 + 2.0 * atanh


def _body(logits_hbm, u_hbm, ca_hbm, probs_hbm, sel_hbm, lv, uv, cav, zv, sv):
    pltpu.sync_copy(logits_hbm, lv)
    pltpu.sync_copy(u_hbm, uv)
    pltpu.sync_copy(ca_hbm, cav)

    lane = lax.iota(jnp.int32, LANES).astype(jnp.float32)

    runmax = jnp.full((LANES,), -3.0e38, jnp.float32)
    for c in range(NCHUNK):
        sl = pl.ds(c * LANES, LANES)
        g = -_log16(-_log16(uv[sl]))
        z = (lv[sl] + g) * TAU_INV
        zv[sl] = z
        runmax = jnp.maximum(runmax, z)
    zmax = jnp.max(runmax)

    acc = jnp.zeros((LANES,), jnp.float32)
    best_e = jnp.full((LANES,), -1.0, jnp.float32)
    best_i = jnp.full((LANES,), 1.0e9, jnp.float32)
    for c in range(NCHUNK):
        sl = pl.ds(c * LANES, LANES)
        e = jnp.exp(zv[sl] - zmax)
        zv[sl] = e
        acc = acc + e
        gidx = lane + float(c * LANES)
        upd = jnp.logical_and(e > best_e, gidx < float(N_ANG))
        best_e = jnp.where(upd, e, best_e)
        best_i = jnp.where(upd, gidx, best_i)
    invv = 1.0 / jnp.full((LANES,), jnp.sum(acc), jnp.float32)

    for c in range(NCHUNK):
        sl = pl.ds(c * LANES, LANES)
        zv[sl] = zv[sl] * invv

    eb = jnp.max(best_e)
    cand = jnp.where(best_e == eb, best_i, 1.0e9)
    hard = jnp.min(cand).astype(jnp.int32)

    idxv = jnp.full((LANES,), hard, jnp.int32)
    sv[...] = plsc.load_gather(cav, [idxv])

    pltpu.sync_copy(zv, probs_hbm)
    pltpu.sync_copy(sv, sel_hbm)


@jax.jit
def kernel(logits, candidate_angles, uniform_noise):
    pad = NPAD - N_ANG
    logits_p = jnp.concatenate([logits, jnp.full((pad,), -1.0e30, jnp.float32)])
    u_p = jnp.concatenate([uniform_noise, jnp.full((pad,), 0.5, jnp.float32)])
    ca_p = jnp.concatenate([candidate_angles, jnp.zeros((pad,), jnp.float32)])

    mesh = plsc.VectorSubcoreMesh(
        core_axis_name="c", subcore_axis_name="s", num_cores=1, num_subcores=1
    )
    probs_p, sel = pl.kernel(
        _body,
        out_type=[
            jax.ShapeDtypeStruct((NPAD,), jnp.float32),
            jax.ShapeDtypeStruct((LANES,), jnp.float32),
        ],
        mesh=mesh,
        compiler_params=pltpu.CompilerParams(needs_layout_passes=False),
        scratch_types=[
            pltpu.VMEM((NPAD,), jnp.float32),
            pltpu.VMEM((NPAD,), jnp.float32),
            pltpu.VMEM((NPAD,), jnp.float32),
            pltpu.VMEM((NPAD,), jnp.float32),
            pltpu.VMEM((LANES,), jnp.float32),
        ],
    )(logits_p, u_p, ca_p)
    return sel[0], probs_p[:N_ANG]
